# Initial kernel scaffold; baseline (speedup 1.0000x reference)
import functools
import math

import numpy as np

import jax
import jax.numpy as jnp
from jax.experimental import pallas as pl
from jax.experimental.pallas import tpu as pltpu

FEAT = 128
N_DATA = 100000
K = 1024
T = 0.07
B = 1024
NP = 102400
TN = 2048
SENT = -4.0

_I32 = jnp.int32

_NEG2_BITS = int(np.float32(-2.0).view(np.int32))
NEG2_KEY = _NEG2_BITS ^ ((_NEG2_BITS >> 31) & 0x7FFFFFFF)


def _f32_sort_key(x):
    b = jax.lax.bitcast_convert_type(x, _I32)
    return b ^ ((b >> 31) & jnp.int32(0x7FFFFFFF))


def _mm_body(a_ref, r_ref, m_ref, pair_ref, thr_ref, s_ref, c_ref, p0_ref):
    j = pl.program_id(0)
    a = a_ref[...]
    r = r_ref[...]
    m = m_ref[...]
    thr = thr_ref[0]

    s = jax.lax.dot_general(a, r, (((1,), (0,)), ((), ())),
                            preferred_element_type=jnp.float32)
    c = jax.lax.dot_general(a, m, (((1,), (1,)), ((), ())),
                            preferred_element_type=jnp.float32)

    col = j * TN + jax.lax.broadcasted_iota(_I32, (B, TN), 1)
    s = jnp.where(s >= thr, jnp.float32(-2.0), s)
    s = jnp.where(col < N_DATA, s, jnp.float32(SENT))
    s_ref[...] = _f32_sort_key(s)
    c_ref[...] = c

    @pl.when(j == 0)
    def _():
        p = pair_ref[...]
        p0_ref[...] = jnp.sum(a * p, axis=1, keepdims=True)


def _mm_stage(anchor, pair, memory_bank, thr_f):
    r = memory_bank.reshape(FEAT, N_DATA)
    grid = (NP // TN,)
    out_shapes = (
        jax.ShapeDtypeStruct((B, NP), _I32),
        jax.ShapeDtypeStruct((B, NP), jnp.float32),
        jax.ShapeDtypeStruct((B, 1), jnp.float32),
    )
    return pl.pallas_call(
        _mm_body,
        grid=grid,
        in_specs=[
            pl.BlockSpec((B, FEAT), lambda j: (0, 0)),
            pl.BlockSpec((FEAT, TN), lambda j: (0, j)),
            pl.BlockSpec((TN, FEAT), lambda j: (j, 0)),
            pl.BlockSpec((B, FEAT), lambda j: (0, 0)),
            pl.BlockSpec(memory_space=pltpu.SMEM),
        ],
        out_specs=(
            pl.BlockSpec((B, TN), lambda j: (0, j)),
            pl.BlockSpec((B, TN), lambda j: (0, j)),
            pl.BlockSpec((B, 1), lambda j: (0, 0)),
        ),
        out_shape=out_shapes,
    )(anchor, r, memory_bank, pair, thr_f)


def _finish_body(v_ref, p0_ref, o_ref):
    p0 = p0_ref[...]
    v = v_ref[...]
    e0 = jnp.exp(p0 / T)
    ev = jnp.exp(v / T)
    total = jnp.sum(e0) + jnp.sum(ev)
    z = total / (B * (K + 1)) * N_DATA
    o_ref[:, 0:1] = e0 / z
    o_ref[:, 1:] = ev / z


def _finish_stage(vals, p0):
    return pl.pallas_call(
        _finish_body,
        out_shape=jax.ShapeDtypeStruct((B, K + 1), jnp.float32),
    )(vals, p0)


def kernel(anchor_feature, pair_feature, membank_idx, threshold, memory_bank):
    thr_f = jnp.asarray(threshold, jnp.float32).reshape(1)
    s_key, c, p0 = _mm_stage(anchor_feature, pair_feature, memory_bank, thr_f)

    top_key, top_idx = jax.lax.top_k(s_key[:, :N_DATA], K)
    rand_idx = jax.random.randint(jax.random.key(1234), (B, K), 0, N_DATA,
                                  dtype=top_idx.dtype)
    final_idx = jnp.where(top_key != NEG2_KEY, top_idx, rand_idx)
    vals = jnp.take_along_axis(c, final_idx, axis=1)

    out = _finish_stage(vals, p0)
    return out.reshape(B, K + 1, 1)


# speedup vs baseline: 1.1593x; 1.1593x over previous
import functools
import math

import numpy as np

import jax
import jax.numpy as jnp
from jax.experimental import pallas as pl
from jax.experimental.pallas import tpu as pltpu

FEAT = 128
N_DATA = 100000
K = 1024
T = 0.07
B = 1024
TN = 2048
NP = 100352
SENT = -4.0

_I32 = jnp.int32

_NEG2_BITS = int(np.float32(-2.0).view(np.int32))
NEG2_KEY = _NEG2_BITS ^ ((_NEG2_BITS >> 31) & 0x7FFFFFFF)


def _f32_sort_key(x):
    b = jax.lax.bitcast_convert_type(x, _I32)
    return b ^ ((b >> 31) & jnp.int32(0x7FFFFFFF))


def _mm_body(a_ref, r_ref, m_ref, pair_ref, thr_ref, s_ref, c_ref, p0_ref):
    j = pl.program_id(0)
    a = a_ref[...]
    r = r_ref[...]
    m = m_ref[...]
    thr = thr_ref[0, 0]

    s = jax.lax.dot_general(a, r, (((1,), (0,)), ((), ())),
                            preferred_element_type=jnp.float32)
    c = jax.lax.dot_general(a, m, (((1,), (1,)), ((), ())),
                            preferred_element_type=jnp.float32)

    col = j * TN + jax.lax.broadcasted_iota(_I32, (B, TN), 1)
    s = jnp.where(s >= thr, jnp.float32(-2.0), s)
    s = jnp.where(col < N_DATA, s, jnp.float32(SENT))
    s_ref[...] = _f32_sort_key(s)
    c_ref[...] = c

    @pl.when(j == 0)
    def _():
        p = pair_ref[...]
        p0_ref[...] = jnp.sum(a * p, axis=1, keepdims=True)


def _mm_stage(anchor, pair, memory_bank, thr_f):
    r = memory_bank.reshape(FEAT, N_DATA)
    grid = (NP // TN,)
    out_shapes = (
        jax.ShapeDtypeStruct((B, NP), _I32),
        jax.ShapeDtypeStruct((B, NP), jnp.float32),
        jax.ShapeDtypeStruct((B, 1), jnp.float32),
    )
    return pl.pallas_call(
        _mm_body,
        grid=grid,
        in_specs=[
            pl.BlockSpec((B, FEAT), lambda j: (0, 0)),
            pl.BlockSpec((FEAT, TN), lambda j: (0, j)),
            pl.BlockSpec((TN, FEAT), lambda j: (j, 0)),
            pl.BlockSpec((B, FEAT), lambda j: (0, 0)),
            pl.BlockSpec(memory_space=pltpu.SMEM),
        ],
        out_specs=(
            pl.BlockSpec((B, TN), lambda j: (0, j)),
            pl.BlockSpec((B, TN), lambda j: (0, j)),
            pl.BlockSpec((B, 1), lambda j: (0, 0)),
        ),
        out_shape=out_shapes,
    )(anchor, r, memory_bank, pair, thr_f)


def _finish_body(v_ref, p0_ref, o_ref):
    p0 = p0_ref[...]
    v = v_ref[...]
    e0 = jnp.exp(p0 / T)
    ev = jnp.exp(v / T)
    total = jnp.sum(e0) + jnp.sum(ev)
    z = total / (B * (K + 1)) * N_DATA
    o_ref[:, 0:1] = e0 / z
    o_ref[:, 1:] = ev / z


def _finish_stage(vals, p0):
    return pl.pallas_call(
        _finish_body,
        out_shape=jax.ShapeDtypeStruct((B, K + 1), jnp.float32),
    )(vals, p0)


def kernel(anchor_feature, pair_feature, membank_idx, threshold, memory_bank):
    thr_f = jnp.asarray(threshold, jnp.float32).reshape(1, 1)
    s_key, c, p0 = _mm_stage(anchor_feature, pair_feature, memory_bank, thr_f)

    top_key, top_idx = jax.lax.top_k(s_key[:, :N_DATA], K)
    rand_idx = jax.random.randint(jax.random.key(1234), (B, K), 0, N_DATA,
                                  dtype=top_idx.dtype)
    final_idx = jnp.where(top_key != NEG2_KEY, top_idx, rand_idx)
    vals = jnp.take_along_axis(c, final_idx, axis=1)

    out = _finish_stage(vals, p0)
    return out.reshape(B, K + 1, 1)


# speedup vs baseline: 11.1504x; 9.6184x over previous
import functools
import math

import numpy as np

import jax
import jax.numpy as jnp
from jax import lax
from jax.experimental import pallas as pl
from jax.experimental.pallas import tpu as pltpu
from jax.experimental.pallas import tpu_sc as plsc

FEAT = 128
N_DATA = 100000
K = 1024
T = 0.07
B = 1024
TN = 2048
NP = 100352
SENT = -4.0

_I32 = jnp.int32

_NEG2_BITS = int(np.float32(-2.0).view(np.int32))
NEG2_KEY = _NEG2_BITS ^ ((_NEG2_BITS >> 31) & 0x7FFFFFFF)


def _f32_sort_key(x):
    b = jax.lax.bitcast_convert_type(x, _I32)
    return b ^ ((b >> 31) & jnp.int32(0x7FFFFFFF))


def _mm_body(a_ref, r_ref, m_ref, pair_ref, thr_ref, s_ref, c_ref, p0_ref):
    j = pl.program_id(0)
    a = a_ref[...]
    r = r_ref[...]
    m = m_ref[...]
    thr = thr_ref[0, 0]

    s = jax.lax.dot_general(a, r, (((1,), (0,)), ((), ())),
                            preferred_element_type=jnp.float32)
    c = jax.lax.dot_general(a, m, (((1,), (1,)), ((), ())),
                            preferred_element_type=jnp.float32)

    col = j * TN + jax.lax.broadcasted_iota(_I32, (B, TN), 1)
    s = jnp.where(s >= thr, jnp.float32(-2.0), s)
    s = jnp.where(col < N_DATA, s, jnp.float32(SENT))
    s_ref[...] = _f32_sort_key(s)
    c_ref[...] = c

    @pl.when(j == 0)
    def _():
        p = pair_ref[...]
        p0_ref[...] = jnp.sum(a * p, axis=1, keepdims=True)


def _mm_stage(anchor, pair, memory_bank, thr_f):
    r = memory_bank.reshape(FEAT, N_DATA)
    grid = (NP // TN,)
    out_shapes = (
        jax.ShapeDtypeStruct((B, NP), _I32),
        jax.ShapeDtypeStruct((B, NP), jnp.float32),
        jax.ShapeDtypeStruct((B, 1), jnp.float32),
    )
    return pl.pallas_call(
        _mm_body,
        grid=grid,
        in_specs=[
            pl.BlockSpec((B, FEAT), lambda j: (0, 0)),
            pl.BlockSpec((FEAT, TN), lambda j: (0, j)),
            pl.BlockSpec((TN, FEAT), lambda j: (j, 0)),
            pl.BlockSpec((B, FEAT), lambda j: (0, 0)),
            pl.BlockSpec(memory_space=pltpu.SMEM),
        ],
        out_specs=(
            pl.BlockSpec((B, TN), lambda j: (0, j)),
            pl.BlockSpec((B, TN), lambda j: (0, j)),
            pl.BlockSpec((B, 1), lambda j: (0, 0)),
        ),
        out_shape=out_shapes,
    )(anchor, r, memory_bank, pair, thr_f)


NW = 32
RPW = B // NW
NBINS = 8192
BIN_SHIFT = 19
CAP = 2048
KINV_NEG2 = NEG2_KEY ^ 0x7FFFFFFF
_ONES16 = lambda: jnp.ones((16,), _I32)


def _sc_body(skey_hbm, cmat_hbm, rand_hbm, out_hbm,
             rowb, hist, ckA, ciA, ckB, ciB, offs, randb, flatb, valsb, sem):
    cid = lax.axis_index("c")
    sid = lax.axis_index("s")
    wid = sid * 2 + cid
    iota = lax.iota(_I32, 16)

    def do_row(i, _):
        r = wid * RPW + i
        pltpu.sync_copy(skey_hbm.at[r], rowb)
        pltpu.sync_copy(rand_hbm.at[r], randb)

        def zero_hist(j, _):
            hist[pl.ds(j * 16, 16)] = jnp.zeros((16,), _I32)
            return 0
        lax.fori_loop(0, NBINS // 16, zero_hist, 0)

        def hist_step(v, _):
            k = rowb[pl.ds(v * 16, 16)]
            b = (k >> BIN_SHIFT) + (NBINS // 2)
            plsc.addupdate_scatter(hist, [b], _ONES16())
            return 0
        lax.fori_loop(0, NP // 16, hist_step, 0)

        target = jnp.int32(NP - K)

        def scan_step(bchunk, carry):
            run, t = carry
            c = hist[pl.ds(bchunk * 16, 16)]
            cs = plsc.cumsum(c)
            cexc = cs - c + run
            cond = cexc <= target
            s = jnp.sum(cond.astype(_I32))
            t = jnp.where(s > 0, bchunk * 16 + s - 1, t)
            run = run + jnp.max(cs)
            return run, t
        _, t = lax.fori_loop(0, NBINS // 16, scan_step,
                             (jnp.int32(0), jnp.int32(0)))
        thr_key = (t - NBINS // 2) << BIN_SHIFT

        def comp_step(v, off):
            k = rowb[pl.ds(v * 16, 16)]
            m = k >= thr_key
            mi = m.astype(_I32)
            csm = plsc.cumsum(mi)
            pos = off + csm - mi
            m2 = m & (pos < CAP)
            plsc.store_scatter(ckA, [pos], k ^ jnp.int32(0x7FFFFFFF), mask=m2)
            plsc.store_scatter(ciA, [pos], v * 16 + iota, mask=m2)
            return off + jnp.take(csm, jnp.full((16,), 15, _I32))
        offv = lax.fori_loop(0, NP // 16, comp_step, jnp.zeros((16,), _I32))
        nc = jnp.minimum(jnp.max(offv), CAP)
        nv = (nc + 15) // 16

        bufs = [(ckA, ciA, ckB, ciB), (ckB, ciB, ckA, ciA)]
        for p in range(5):
            ink, ini, outk, outi = bufs[p % 2]
            sh = 7 * p
            for bq in range(8):
                offs[pl.ds(bq * 16, 16)] = jnp.zeros((16,), _I32)

            def cnt_step(v, _, ink=ink, sh=sh):
                kk = ink[pl.ds(v * 16, 16)]
                gm = (v * 16 + iota) < nc
                d = lax.shift_right_logical(kk, sh) & 127
                plsc.addupdate_scatter(offs, [d], _ONES16(), mask=gm)
                return 0
            lax.fori_loop(0, nv, cnt_step, 0)

            run = jnp.int32(0)
            for bq in range(8):
                c = offs[pl.ds(bq * 16, 16)]
                cs = plsc.cumsum(c)
                offs[pl.ds(bq * 16, 16)] = cs - c + run
                run = run + jnp.max(cs)

            def perm_step(v, _, ink=ink, ini=ini, outk=outk, outi=outi, sh=sh):
                kk = ink[pl.ds(v * 16, 16)]
                vi = ini[pl.ds(v * 16, 16)]
                gm = (v * 16 + iota) < nc
                d = lax.shift_right_logical(kk, sh) & 127
                sk, sv, sm = plsc.sort_key_val(d, iota, mask=gm)
                prev = jnp.take(sk, jnp.maximum(iota - 1, 0))
                is_start = (iota == 0) | (sk != prev)
                base = plsc.cummax(jnp.where(is_start, iota, 0))
                rank = iota - base
                og = plsc.load_gather(offs, [sk], mask=sm)
                pos = og + rank
                kks = jnp.take(kk, sv)
                vis = jnp.take(vi, sv)
                plsc.store_scatter(outk, [pos], kks, mask=sm)
                plsc.store_scatter(outi, [pos], vis, mask=sm)
                plsc.addupdate_scatter(offs, [sk], _ONES16(), mask=sm)
                return 0
            lax.fori_loop(0, nv, perm_step, 0)

        def sel_step(v, _):
            kv = ckB[pl.ds(v * 16, 16)]
            ivv = ciB[pl.ds(v * 16, 16)]
            rv = randb[pl.ds(v * 16, 16)]
            sel = kv != jnp.int32(KINV_NEG2)
            flatb[pl.ds(v * 16, 16)] = r * NP + jnp.where(sel, ivv, rv)
            return 0
        lax.fori_loop(0, K // 16, sel_step, 0)

        pltpu.async_copy(cmat_hbm.at[flatb], valsb, sem).wait()
        pltpu.sync_copy(valsb, out_hbm.at[r])
        return 0

    lax.fori_loop(0, RPW, do_row, 0)


def _sc_topk(s_key, c, rand_idx):
    mesh = plsc.VectorSubcoreMesh(core_axis_name="c", subcore_axis_name="s")
    f = functools.partial(
        pl.kernel,
        out_type=jax.ShapeDtypeStruct((B, K), jnp.float32),
        mesh=mesh,
        scratch_types=[
            pltpu.VMEM((NP,), _I32),
            pltpu.VMEM((NBINS,), _I32),
            pltpu.VMEM((CAP + 16,), _I32),
            pltpu.VMEM((CAP + 16,), _I32),
            pltpu.VMEM((CAP + 16,), _I32),
            pltpu.VMEM((CAP + 16,), _I32),
            pltpu.VMEM((128,), _I32),
            pltpu.VMEM((K,), _I32),
            pltpu.VMEM((K,), _I32),
            pltpu.VMEM((K,), jnp.float32),
            pltpu.SemaphoreType.DMA,
        ],
        compiler_params=pltpu.CompilerParams(needs_layout_passes=False),
    )(_sc_body)
    return f(s_key, c.reshape(B * NP), rand_idx)


def _finish_body(v_ref, p0_ref, o_ref):
    p0 = p0_ref[...]
    v = v_ref[...]
    e0 = jnp.exp(p0 / T)
    ev = jnp.exp(v / T)
    total = jnp.sum(e0) + jnp.sum(ev)
    z = total / (B * (K + 1)) * N_DATA
    o_ref[:, 0:1] = e0 / z
    o_ref[:, 1:] = ev / z


def _finish_stage(vals, p0):
    return pl.pallas_call(
        _finish_body,
        out_shape=jax.ShapeDtypeStruct((B, K + 1), jnp.float32),
    )(vals, p0)


def kernel(anchor_feature, pair_feature, membank_idx, threshold, memory_bank):
    thr_f = jnp.asarray(threshold, jnp.float32).reshape(1, 1)
    s_key, c, p0 = _mm_stage(anchor_feature, pair_feature, memory_bank, thr_f)

    rand_idx = jax.random.randint(jax.random.key(1234), (B, K), 0, N_DATA,
                                  dtype=jnp.int32)
    vals = _sc_topk(s_key, c, rand_idx)

    out = _finish_stage(vals, p0)
    return out.reshape(B, K + 1, 1)


# speedup vs baseline: 11.1599x; 1.0009x over previous
import functools
import math

import numpy as np

import jax
import jax.numpy as jnp
from jax import lax
from jax.experimental import pallas as pl
from jax.experimental.pallas import tpu as pltpu
from jax.experimental.pallas import tpu_sc as plsc

FEAT = 128
N_DATA = 100000
K = 1024
T = 0.07
B = 1024
TN = 2048
NP = 100352
SENT = -4.0

_I32 = jnp.int32

_NEG2_BITS = int(np.float32(-2.0).view(np.int32))
NEG2_KEY = _NEG2_BITS ^ ((_NEG2_BITS >> 31) & 0x7FFFFFFF)


def _f32_sort_key(x):
    b = jax.lax.bitcast_convert_type(x, _I32)
    return b ^ ((b >> 31) & jnp.int32(0x7FFFFFFF))


def _mm_body(a_ref, r_ref, m_ref, pair_ref, thr_ref, s_ref, c_ref, p0_ref):
    j = pl.program_id(0)
    a = a_ref[...]
    r = r_ref[...]
    m = m_ref[...]
    thr = thr_ref[0, 0]

    s = jax.lax.dot_general(a, r, (((1,), (0,)), ((), ())),
                            preferred_element_type=jnp.float32)
    c = jax.lax.dot_general(a, m, (((1,), (1,)), ((), ())),
                            preferred_element_type=jnp.float32)

    col = j * TN + jax.lax.broadcasted_iota(_I32, (B, TN), 1)
    s = jnp.where(s >= thr, jnp.float32(-2.0), s)
    s = jnp.where(col < N_DATA, s, jnp.float32(SENT))
    s_ref[...] = _f32_sort_key(s)
    c_ref[...] = c

    @pl.when(j == 0)
    def _():
        p = pair_ref[...]
        p0_ref[...] = jnp.sum(a * p, axis=1, keepdims=True)


def _mm_stage(anchor, pair, memory_bank, thr_f):
    r = memory_bank.reshape(FEAT, N_DATA)
    grid = (NP // TN,)
    out_shapes = (
        jax.ShapeDtypeStruct((B, NP), _I32),
        jax.ShapeDtypeStruct((B, NP), jnp.float32),
        jax.ShapeDtypeStruct((B, 1), jnp.float32),
    )
    return pl.pallas_call(
        _mm_body,
        grid=grid,
        in_specs=[
            pl.BlockSpec((B, FEAT), lambda j: (0, 0)),
            pl.BlockSpec((FEAT, TN), lambda j: (0, j)),
            pl.BlockSpec((TN, FEAT), lambda j: (j, 0)),
            pl.BlockSpec((B, FEAT), lambda j: (0, 0)),
            pl.BlockSpec(memory_space=pltpu.SMEM),
        ],
        out_specs=(
            pl.BlockSpec((B, TN), lambda j: (0, j)),
            pl.BlockSpec((B, TN), lambda j: (0, j)),
            pl.BlockSpec((B, 1), lambda j: (0, 0)),
        ),
        out_shape=out_shapes,
    )(anchor, r, memory_bank, pair, thr_f)


NW = 32
RPW = B // NW
NBINS = 8192
BIN_SHIFT = 19
CAP = 2048
KINV_NEG2 = NEG2_KEY ^ 0x7FFFFFFF
_ONES16 = lambda: jnp.ones((16,), _I32)


def _sc_body(skey_hbm, cmat_hbm, rand_hbm, out_hbm,
             rowb, hist, ckA, ciA, ckB, ciB, offs, randb, flatb, valsb, sem):
    cid = lax.axis_index("c")
    sid = lax.axis_index("s")
    wid = sid * 2 + cid
    iota = lax.iota(_I32, 16)

    def do_row(i, _):
        r = wid * RPW + i
        pltpu.sync_copy(skey_hbm.at[r], rowb)
        pltpu.sync_copy(rand_hbm.at[r], randb)

        with jax.named_scope("zero_hist"):
            def zero_hist(j, _):
                hist[pl.ds(j * 16, 16)] = jnp.zeros((16,), _I32)
                return 0
            lax.fori_loop(0, NBINS // 16, zero_hist, 0)

        with jax.named_scope("hist"):
            def hist_step(v, _):
                k = rowb[pl.ds(v * 16, 16)]
                b = (k >> BIN_SHIFT) + (NBINS // 2)
                plsc.addupdate_scatter(hist, [b], _ONES16())
                return 0
            lax.fori_loop(0, NP // 16, hist_step, 0)

        target = jnp.int32(NP - K)

        with jax.named_scope("thresh_scan"):
            def scan_step(bchunk, carry):
                run, t = carry
                c = hist[pl.ds(bchunk * 16, 16)]
                cs = plsc.cumsum(c)
                cexc = cs - c + run
                cond = cexc <= target
                s = jnp.sum(cond.astype(_I32))
                t = jnp.where(s > 0, bchunk * 16 + s - 1, t)
                run = run + jnp.max(cs)
                return run, t
            _, t = lax.fori_loop(0, NBINS // 16, scan_step,
                                 (jnp.int32(0), jnp.int32(0)))
            thr_key = (t - NBINS // 2) << BIN_SHIFT

        with jax.named_scope("compact"):
            def comp_step(v, off):
                k = rowb[pl.ds(v * 16, 16)]
                m = k >= thr_key
                mi = m.astype(_I32)
                csm = plsc.cumsum(mi)
                pos = off + csm - mi
                m2 = m & (pos < CAP)
                plsc.store_scatter(ckA, [pos], k ^ jnp.int32(0x7FFFFFFF),
                                   mask=m2)
                plsc.store_scatter(ciA, [pos], v * 16 + iota, mask=m2)
                return off + jnp.take(csm, jnp.full((16,), 15, _I32))
            offv = lax.fori_loop(0, NP // 16, comp_step,
                                 jnp.zeros((16,), _I32))
            nc = jnp.minimum(jnp.max(offv), CAP)
            nv = (nc + 15) // 16

        bufs = [(ckA, ciA, ckB, ciB), (ckB, ciB, ckA, ciA)]
        rscope = jax.named_scope("radix")
        rscope.__enter__()
        for p in range(5):
            ink, ini, outk, outi = bufs[p % 2]
            sh = 7 * p
            for bq in range(8):
                offs[pl.ds(bq * 16, 16)] = jnp.zeros((16,), _I32)

            def cnt_step(v, _, ink=ink, sh=sh):
                kk = ink[pl.ds(v * 16, 16)]
                gm = (v * 16 + iota) < nc
                d = lax.shift_right_logical(kk, sh) & 127
                plsc.addupdate_scatter(offs, [d], _ONES16(), mask=gm)
                return 0
            lax.fori_loop(0, nv, cnt_step, 0)

            run = jnp.int32(0)
            for bq in range(8):
                c = offs[pl.ds(bq * 16, 16)]
                cs = plsc.cumsum(c)
                offs[pl.ds(bq * 16, 16)] = cs - c + run
                run = run + jnp.max(cs)

            def perm_step(v, _, ink=ink, ini=ini, outk=outk, outi=outi, sh=sh):
                kk = ink[pl.ds(v * 16, 16)]
                vi = ini[pl.ds(v * 16, 16)]
                gm = (v * 16 + iota) < nc
                d = lax.shift_right_logical(kk, sh) & 127
                sk, sv, sm = plsc.sort_key_val(d, iota, mask=gm)
                prev = jnp.take(sk, jnp.maximum(iota - 1, 0))
                is_start = (iota == 0) | (sk != prev)
                base = plsc.cummax(jnp.where(is_start, iota, 0))
                rank = iota - base
                og = plsc.load_gather(offs, [sk], mask=sm)
                pos = og + rank
                kks = jnp.take(kk, sv)
                vis = jnp.take(vi, sv)
                plsc.store_scatter(outk, [pos], kks, mask=sm)
                plsc.store_scatter(outi, [pos], vis, mask=sm)
                plsc.addupdate_scatter(offs, [sk], _ONES16(), mask=sm)
                return 0
            lax.fori_loop(0, nv, perm_step, 0)
        rscope.__exit__(None, None, None)

        def sel_step(v, _):
            kv = ckB[pl.ds(v * 16, 16)]
            ivv = ciB[pl.ds(v * 16, 16)]
            rv = randb[pl.ds(v * 16, 16)]
            sel = kv != jnp.int32(KINV_NEG2)
            flatb[pl.ds(v * 16, 16)] = r * NP + jnp.where(sel, ivv, rv)
            return 0
        lax.fori_loop(0, K // 16, sel_step, 0)

        pltpu.async_copy(cmat_hbm.at[flatb], valsb, sem).wait()
        pltpu.sync_copy(valsb, out_hbm.at[r])
        return 0

    lax.fori_loop(0, RPW, do_row, 0)


def _sc_topk(s_key, c, rand_idx):
    mesh = plsc.VectorSubcoreMesh(core_axis_name="c", subcore_axis_name="s")
    f = functools.partial(
        pl.kernel,
        out_type=jax.ShapeDtypeStruct((B, K), jnp.float32),
        mesh=mesh,
        scratch_types=[
            pltpu.VMEM((NP,), _I32),
            pltpu.VMEM((NBINS,), _I32),
            pltpu.VMEM((CAP + 16,), _I32),
            pltpu.VMEM((CAP + 16,), _I32),
            pltpu.VMEM((CAP + 16,), _I32),
            pltpu.VMEM((CAP + 16,), _I32),
            pltpu.VMEM((128,), _I32),
            pltpu.VMEM((K,), _I32),
            pltpu.VMEM((K,), _I32),
            pltpu.VMEM((K,), jnp.float32),
            pltpu.SemaphoreType.DMA,
        ],
        compiler_params=pltpu.CompilerParams(needs_layout_passes=False),
    )(_sc_body)
    return f(s_key, c.reshape(B * NP), rand_idx)


def _finish_body(v_ref, p0_ref, o_ref):
    p0 = p0_ref[...]
    v = v_ref[...]
    e0 = jnp.exp(p0 / T)
    ev = jnp.exp(v / T)
    total = jnp.sum(e0) + jnp.sum(ev)
    z = total / (B * (K + 1)) * N_DATA
    o_ref[:, 0:1] = e0 / z
    o_ref[:, 1:] = ev / z


def _finish_stage(vals, p0):
    return pl.pallas_call(
        _finish_body,
        out_shape=jax.ShapeDtypeStruct((B, K + 1), jnp.float32),
    )(vals, p0)


def kernel(anchor_feature, pair_feature, membank_idx, threshold, memory_bank):
    thr_f = jnp.asarray(threshold, jnp.float32).reshape(1, 1)
    s_key, c, p0 = _mm_stage(anchor_feature, pair_feature, memory_bank, thr_f)

    rand_idx = jax.random.randint(jax.random.key(1234), (B, K), 0, N_DATA,
                                  dtype=jnp.int32)
    vals = _sc_topk(s_key, c, rand_idx)

    out = _finish_stage(vals, p0)
    return out.reshape(B, K + 1, 1)


# speedup vs baseline: 25.0254x; 2.2424x over previous
import functools
import math

import numpy as np

import jax
import jax.numpy as jnp
from jax import lax
from jax.experimental import pallas as pl
from jax.experimental.pallas import tpu as pltpu
from jax.experimental.pallas import tpu_sc as plsc

FEAT = 128
N_DATA = 100000
K = 1024
T = 0.07
B = 1024
TN = 2048
NP = 100352
SENT = -4.0

_I32 = jnp.int32

_NEG2_BITS = int(np.float32(-2.0).view(np.int32))
NEG2_KEY = _NEG2_BITS ^ ((_NEG2_BITS >> 31) & 0x7FFFFFFF)


def _f32_sort_key(x):
    b = jax.lax.bitcast_convert_type(x, _I32)
    return b ^ ((b >> 31) & jnp.int32(0x7FFFFFFF))


def _mm_body(a_ref, r_ref, m_ref, pair_ref, thr_ref, s_ref, c_ref, p0_ref):
    j = pl.program_id(0)
    a = a_ref[...]
    r = r_ref[...]
    m = m_ref[...]
    thr = thr_ref[0, 0]

    s = jax.lax.dot_general(a, r, (((1,), (0,)), ((), ())),
                            preferred_element_type=jnp.float32)
    c = jax.lax.dot_general(a, m, (((1,), (1,)), ((), ())),
                            preferred_element_type=jnp.float32)

    col = j * TN + jax.lax.broadcasted_iota(_I32, (B, TN), 1)
    s = jnp.where(s >= thr, jnp.float32(-2.0), s)
    s = jnp.where(col < N_DATA, s, jnp.float32(SENT))
    s_ref[...] = _f32_sort_key(s)
    c_ref[...] = c

    @pl.when(j == 0)
    def _():
        p = pair_ref[...]
        p0_ref[...] = jnp.sum(a * p, axis=1, keepdims=True)


def _mm_stage(anchor, pair, memory_bank, thr_f):
    r = memory_bank.reshape(FEAT, N_DATA)
    grid = (NP // TN,)
    out_shapes = (
        jax.ShapeDtypeStruct((B, NP), _I32),
        jax.ShapeDtypeStruct((B, NP), jnp.float32),
        jax.ShapeDtypeStruct((B, 1), jnp.float32),
    )
    return pl.pallas_call(
        _mm_body,
        grid=grid,
        in_specs=[
            pl.BlockSpec((B, FEAT), lambda j: (0, 0)),
            pl.BlockSpec((FEAT, TN), lambda j: (0, j)),
            pl.BlockSpec((TN, FEAT), lambda j: (j, 0)),
            pl.BlockSpec((B, FEAT), lambda j: (0, 0)),
            pl.BlockSpec(memory_space=pltpu.SMEM),
        ],
        out_specs=(
            pl.BlockSpec((B, TN), lambda j: (0, j)),
            pl.BlockSpec((B, TN), lambda j: (0, j)),
            pl.BlockSpec((B, 1), lambda j: (0, 0)),
        ),
        out_shape=out_shapes,
    )(anchor, r, memory_bank, pair, thr_f)


NW = 32
RPW = B // NW
NBINS = 8192
BIN_SHIFT = 19
CAP = 2048
KINV_NEG2 = NEG2_KEY ^ 0x7FFFFFFF
_ONES16 = lambda: jnp.ones((16,), _I32)


def _sc_body(skey_hbm, cmat_hbm, rand_hbm, out_hbm,
             rowb, hist, ckA, ciA, ckB, ciB, offs, randb, flatb, valsb, sem):
    cid = lax.axis_index("c")
    sid = lax.axis_index("s")
    wid = sid * 2 + cid
    iota = lax.iota(_I32, 16)

    def do_row(i, _):
        r = wid * RPW + i
        pltpu.sync_copy(skey_hbm.at[r], rowb)
        pltpu.sync_copy(rand_hbm.at[r], randb)

        with jax.named_scope("zero_hist"):
            @plsc.parallel_loop(0, NBINS // 16, 1, unroll=8)
            def _(j):
                hist[pl.ds(j * 16, 16)] = jnp.zeros((16,), _I32)

        with jax.named_scope("hist"):
            @plsc.parallel_loop(0, NP // 16, 1, unroll=8)
            def _(v):
                k = rowb[pl.ds(v * 16, 16)]
                b = (k >> BIN_SHIFT) + (NBINS // 2)
                plsc.addupdate_scatter(hist, [b], _ONES16())

        target = jnp.int32(NP - K)

        with jax.named_scope("thresh_scan"):
            def scan_step(bchunk, carry):
                run, t = carry
                c = hist[pl.ds(bchunk * 16, 16)]
                cs = plsc.cumsum(c)
                cexc = cs - c + run
                cond = cexc <= target
                s = jnp.sum(cond.astype(_I32))
                t = jnp.where(s > 0, bchunk * 16 + s - 1, t)
                run = run + jnp.max(cs)
                return run, t
            _, t = lax.fori_loop(0, NBINS // 16, scan_step,
                                 (jnp.int32(0), jnp.int32(0)))
            thr_key = (t - NBINS // 2) << BIN_SHIFT

        with jax.named_scope("compact"):
            lane15 = jnp.full((16,), 15, _I32)

            def comp_step(g, off):
                ks, csms, mis = [], [], []
                for j in range(4):
                    k = rowb[pl.ds((g * 4 + j) * 16, 16)]
                    m = k >= thr_key
                    mi = m.astype(_I32)
                    ks.append(k)
                    mis.append(mi)
                    csms.append(plsc.cumsum(mi))
                for j in range(4):
                    pos = off + csms[j] - mis[j]
                    m2 = (mis[j] > 0) & (pos < CAP)
                    plsc.store_scatter(ckA, [pos],
                                       ks[j] ^ jnp.int32(0x7FFFFFFF),
                                       mask=m2)
                    plsc.store_scatter(ciA, [pos],
                                       (g * 4 + j) * 16 + iota, mask=m2)
                    off = off + jnp.take(csms[j], lane15)
                return off
            offv = lax.fori_loop(0, NP // 64, comp_step,
                                 jnp.zeros((16,), _I32))
            nc = jnp.minimum(jnp.max(offv), CAP)
            nv = (nc + 15) // 16

        bufs = [(ckA, ciA, ckB, ciB), (ckB, ciB, ckA, ciA)]
        rscope = jax.named_scope("radix")
        rscope.__enter__()
        for p in range(5):
            ink, ini, outk, outi = bufs[p % 2]
            sh = 7 * p
            for bq in range(8):
                offs[pl.ds(bq * 16, 16)] = jnp.zeros((16,), _I32)

            def cnt_step(v, _, ink=ink, sh=sh):
                kk = ink[pl.ds(v * 16, 16)]
                gm = (v * 16 + iota) < nc
                d = lax.shift_right_logical(kk, sh) & 127
                plsc.addupdate_scatter(offs, [d], _ONES16(), mask=gm)
                return 0
            lax.fori_loop(0, nv, cnt_step, 0)

            run = jnp.int32(0)
            for bq in range(8):
                c = offs[pl.ds(bq * 16, 16)]
                cs = plsc.cumsum(c)
                offs[pl.ds(bq * 16, 16)] = cs - c + run
                run = run + jnp.max(cs)

            def perm_step(v, _, ink=ink, ini=ini, outk=outk, outi=outi, sh=sh):
                kk = ink[pl.ds(v * 16, 16)]
                vi = ini[pl.ds(v * 16, 16)]
                gm = (v * 16 + iota) < nc
                d = lax.shift_right_logical(kk, sh) & 127
                sk, sv, sm = plsc.sort_key_val(d, iota, mask=gm)
                prev = jnp.take(sk, jnp.maximum(iota - 1, 0))
                is_start = (iota == 0) | (sk != prev)
                base = plsc.cummax(jnp.where(is_start, iota, 0))
                rank = iota - base
                og = plsc.load_gather(offs, [sk], mask=sm)
                pos = og + rank
                kks = jnp.take(kk, sv)
                vis = jnp.take(vi, sv)
                plsc.store_scatter(outk, [pos], kks, mask=sm)
                plsc.store_scatter(outi, [pos], vis, mask=sm)
                plsc.addupdate_scatter(offs, [sk], _ONES16(), mask=sm)
                return 0
            lax.fori_loop(0, nv, perm_step, 0)
        rscope.__exit__(None, None, None)

        def sel_step(v, _):
            kv = ckB[pl.ds(v * 16, 16)]
            ivv = ciB[pl.ds(v * 16, 16)]
            rv = randb[pl.ds(v * 16, 16)]
            sel = kv != jnp.int32(KINV_NEG2)
            flatb[pl.ds(v * 16, 16)] = r * NP + jnp.where(sel, ivv, rv)
            return 0
        lax.fori_loop(0, K // 16, sel_step, 0)

        pltpu.async_copy(cmat_hbm.at[flatb], valsb, sem).wait()
        pltpu.sync_copy(valsb, out_hbm.at[r])
        return 0

    lax.fori_loop(0, RPW, do_row, 0)


def _sc_topk(s_key, c, rand_idx):
    mesh = plsc.VectorSubcoreMesh(core_axis_name="c", subcore_axis_name="s")
    f = functools.partial(
        pl.kernel,
        out_type=jax.ShapeDtypeStruct((B, K), jnp.float32),
        mesh=mesh,
        scratch_types=[
            pltpu.VMEM((NP,), _I32),
            pltpu.VMEM((NBINS,), _I32),
            pltpu.VMEM((CAP + 16,), _I32),
            pltpu.VMEM((CAP + 16,), _I32),
            pltpu.VMEM((CAP + 16,), _I32),
            pltpu.VMEM((CAP + 16,), _I32),
            pltpu.VMEM((128,), _I32),
            pltpu.VMEM((K,), _I32),
            pltpu.VMEM((K,), _I32),
            pltpu.VMEM((K,), jnp.float32),
            pltpu.SemaphoreType.DMA,
        ],
        compiler_params=pltpu.CompilerParams(needs_layout_passes=False),
    )(_sc_body)
    return f(s_key, c.reshape(B * NP), rand_idx)


def _finish_body(v_ref, p0_ref, o_ref):
    p0 = p0_ref[...]
    v = v_ref[...]
    e0 = jnp.exp(p0 / T)
    ev = jnp.exp(v / T)
    total = jnp.sum(e0) + jnp.sum(ev)
    z = total / (B * (K + 1)) * N_DATA
    o_ref[:, 0:1] = e0 / z
    o_ref[:, 1:] = ev / z


def _finish_stage(vals, p0):
    return pl.pallas_call(
        _finish_body,
        out_shape=jax.ShapeDtypeStruct((B, K + 1), jnp.float32),
    )(vals, p0)


def kernel(anchor_feature, pair_feature, membank_idx, threshold, memory_bank):
    thr_f = jnp.asarray(threshold, jnp.float32).reshape(1, 1)
    s_key, c, p0 = _mm_stage(anchor_feature, pair_feature, memory_bank, thr_f)

    rand_idx = jax.random.randint(jax.random.key(1234), (B, K), 0, N_DATA,
                                  dtype=jnp.int32)
    vals = _sc_topk(s_key, c, rand_idx)

    out = _finish_stage(vals, p0)
    return out.reshape(B, K + 1, 1)


# speedup vs baseline: 31.1272x; 1.2438x over previous
import functools
import math

import numpy as np

import jax
import jax.numpy as jnp
from jax import lax
from jax.experimental import pallas as pl
from jax.experimental.pallas import tpu as pltpu
from jax.experimental.pallas import tpu_sc as plsc

FEAT = 128
N_DATA = 100000
K = 1024
T = 0.07
B = 1024
TN = 2048
NP = 100352
SENT = -4.0

_I32 = jnp.int32

_NEG2_BITS = int(np.float32(-2.0).view(np.int32))
NEG2_KEY = _NEG2_BITS ^ ((_NEG2_BITS >> 31) & 0x7FFFFFFF)


def _f32_sort_key(x):
    b = jax.lax.bitcast_convert_type(x, _I32)
    return b ^ ((b >> 31) & jnp.int32(0x7FFFFFFF))


def _mm_body(a_ref, r_ref, m_ref, pair_ref, thr_ref, s_ref, c_ref, p0_ref):
    j = pl.program_id(0)
    a = a_ref[...]
    r = r_ref[...]
    m = m_ref[...]
    thr = thr_ref[0, 0]

    s = jax.lax.dot_general(a, r, (((1,), (0,)), ((), ())),
                            preferred_element_type=jnp.float32)
    c = jax.lax.dot_general(a, m, (((1,), (1,)), ((), ())),
                            preferred_element_type=jnp.float32)

    col = j * TN + jax.lax.broadcasted_iota(_I32, (B, TN), 1)
    s = jnp.where(s >= thr, jnp.float32(-2.0), s)
    s = jnp.where(col < N_DATA, s, jnp.float32(SENT))
    s_ref[...] = _f32_sort_key(s)
    c_ref[...] = c

    @pl.when(j == 0)
    def _():
        p = pair_ref[...]
        p0_ref[...] = jnp.sum(a * p, axis=1, keepdims=True)


def _mm_stage(anchor, pair, memory_bank, thr_f):
    r = memory_bank.reshape(FEAT, N_DATA)
    grid = (NP // TN,)
    out_shapes = (
        jax.ShapeDtypeStruct((B, NP), _I32),
        jax.ShapeDtypeStruct((B, NP), jnp.float32),
        jax.ShapeDtypeStruct((B, 1), jnp.float32),
    )
    return pl.pallas_call(
        _mm_body,
        grid=grid,
        in_specs=[
            pl.BlockSpec((B, FEAT), lambda j: (0, 0)),
            pl.BlockSpec((FEAT, TN), lambda j: (0, j)),
            pl.BlockSpec((TN, FEAT), lambda j: (j, 0)),
            pl.BlockSpec((B, FEAT), lambda j: (0, 0)),
            pl.BlockSpec(memory_space=pltpu.SMEM),
        ],
        out_specs=(
            pl.BlockSpec((B, TN), lambda j: (0, j)),
            pl.BlockSpec((B, TN), lambda j: (0, j)),
            pl.BlockSpec((B, 1), lambda j: (0, 0)),
        ),
        out_shape=out_shapes,
    )(anchor, r, memory_bank, pair, thr_f)


NW = 32
RPW = B // NW
NBINS = 8192
BIN_SHIFT = 19
CAP = 2048
KINV_NEG2 = NEG2_KEY ^ 0x7FFFFFFF
_ONES16 = lambda: jnp.ones((16,), _I32)


def _sc_body(skey_hbm, cmat_hbm, rand_hbm, out_hbm,
             rowb, hist, ckA, ciA, ckB, ciB, offs, randb, flatb, valsb,
             sem, sem2):
    cid = lax.axis_index("c")
    sid = lax.axis_index("s")
    wid = sid * 2 + cid
    iota = lax.iota(_I32, 16)

    pltpu.async_copy(skey_hbm.at[wid * RPW], rowb, sem2)

    def do_row(i, _):
        r = wid * RPW + i
        pltpu.make_async_copy(skey_hbm.at[r], rowb, sem2).wait()
        pltpu.sync_copy(rand_hbm.at[r], randb)

        with jax.named_scope("zero_hist"):
            @plsc.parallel_loop(0, NBINS // 16, 1, unroll=8)
            def _(j):
                hist[pl.ds(j * 16, 16)] = jnp.zeros((16,), _I32)

        with jax.named_scope("hist"):
            @plsc.parallel_loop(0, NP // 16, 1, unroll=8)
            def _(v):
                k = rowb[pl.ds(v * 16, 16)]
                b = (k >> BIN_SHIFT) + (NBINS // 2)
                plsc.addupdate_scatter(hist, [b], _ONES16())

        target = jnp.int32(NP - K)

        with jax.named_scope("thresh_scan"):
            def scan_step(bchunk, carry):
                run, t = carry
                c = hist[pl.ds(bchunk * 16, 16)]
                cs = plsc.cumsum(c)
                cexc = cs - c + run
                cond = cexc <= target
                s = jnp.sum(cond.astype(_I32))
                t = jnp.where(s > 0, bchunk * 16 + s - 1, t)
                run = run + jnp.max(cs)
                return run, t
            _, t = lax.fori_loop(0, NBINS // 16, scan_step,
                                 (jnp.int32(0), jnp.int32(0)))
            thr_key = (t - NBINS // 2) << BIN_SHIFT

        with jax.named_scope("compact"):
            lane15 = jnp.full((16,), 15, _I32)

            def comp_step(g, off):
                ks, csms, mis = [], [], []
                for j in range(8):
                    k = rowb[pl.ds((g * 8 + j) * 16, 16)]
                    m = k >= thr_key
                    mi = m.astype(_I32)
                    ks.append(k)
                    mis.append(mi)
                    csms.append(plsc.cumsum(mi))
                for j in range(8):
                    pos = off + csms[j] - mis[j]
                    m2 = (mis[j] > 0) & (pos < CAP)
                    plsc.store_scatter(ckA, [pos],
                                       ks[j] ^ jnp.int32(0x7FFFFFFF),
                                       mask=m2)
                    plsc.store_scatter(ciA, [pos],
                                       (g * 8 + j) * 16 + iota, mask=m2)
                    off = off + jnp.take(csms[j], lane15)
                return off
            offv = lax.fori_loop(0, NP // 128, comp_step,
                                 jnp.zeros((16,), _I32))
            nc = jnp.minimum(jnp.max(offv), CAP)
            nv = (nc + 15) // 16

        @pl.when(i + 1 < RPW)
        def _():
            pltpu.async_copy(skey_hbm.at[r + 1], rowb, sem2)

        bufs = [(ckA, ciA, ckB, ciB), (ckB, ciB, ckA, ciA)]
        rscope = jax.named_scope("radix")
        rscope.__enter__()
        for p in range(5):
            ink, ini, outk, outi = bufs[p % 2]
            sh = 7 * p
            for bq in range(8):
                offs[pl.ds(bq * 16, 16)] = jnp.zeros((16,), _I32)

            @plsc.parallel_loop(0, nv, 1, unroll=4)
            def _(v, ink=ink, sh=sh):
                kk = ink[pl.ds(v * 16, 16)]
                gm = (v * 16 + iota) < nc
                d = lax.shift_right_logical(kk, sh) & 127
                plsc.addupdate_scatter(offs, [d], _ONES16(), mask=gm)

            run = jnp.int32(0)
            for bq in range(8):
                c = offs[pl.ds(bq * 16, 16)]
                cs = plsc.cumsum(c)
                offs[pl.ds(bq * 16, 16)] = cs - c + run
                run = run + jnp.max(cs)

            def perm_step(v, _, ink=ink, ini=ini, outk=outk, outi=outi, sh=sh):
                kk = ink[pl.ds(v * 16, 16)]
                vi = ini[pl.ds(v * 16, 16)]
                gm = (v * 16 + iota) < nc
                d = lax.shift_right_logical(kk, sh) & 127
                sk, sv, sm = plsc.sort_key_val(d, iota, mask=gm)
                prev = jnp.take(sk, jnp.maximum(iota - 1, 0))
                is_start = (iota == 0) | (sk != prev)
                base = plsc.cummax(jnp.where(is_start, iota, 0))
                rank = iota - base
                og = plsc.load_gather(offs, [sk], mask=sm)
                pos = og + rank
                kks = jnp.take(kk, sv)
                vis = jnp.take(vi, sv)
                plsc.store_scatter(outk, [pos], kks, mask=sm)
                plsc.store_scatter(outi, [pos], vis, mask=sm)
                plsc.addupdate_scatter(offs, [sk], _ONES16(), mask=sm)
                return 0
            lax.fori_loop(0, nv, perm_step, 0)
        rscope.__exit__(None, None, None)

        def sel_step(v, _):
            kv = ckB[pl.ds(v * 16, 16)]
            ivv = ciB[pl.ds(v * 16, 16)]
            rv = randb[pl.ds(v * 16, 16)]
            sel = kv != jnp.int32(KINV_NEG2)
            flatb[pl.ds(v * 16, 16)] = r * NP + jnp.where(sel, ivv, rv)
            return 0
        lax.fori_loop(0, K // 16, sel_step, 0)

        pltpu.async_copy(cmat_hbm.at[flatb], valsb, sem).wait()
        pltpu.sync_copy(valsb, out_hbm.at[r])
        return 0

    lax.fori_loop(0, RPW, do_row, 0)


def _sc_topk(s_key, c, rand_idx):
    mesh = plsc.VectorSubcoreMesh(core_axis_name="c", subcore_axis_name="s")
    f = functools.partial(
        pl.kernel,
        out_type=jax.ShapeDtypeStruct((B, K), jnp.float32),
        mesh=mesh,
        scratch_types=[
            pltpu.VMEM((NP,), _I32),
            pltpu.VMEM((NBINS,), _I32),
            pltpu.VMEM((CAP + 16,), _I32),
            pltpu.VMEM((CAP + 16,), _I32),
            pltpu.VMEM((CAP + 16,), _I32),
            pltpu.VMEM((CAP + 16,), _I32),
            pltpu.VMEM((128,), _I32),
            pltpu.VMEM((K,), _I32),
            pltpu.VMEM((K,), _I32),
            pltpu.VMEM((K,), jnp.float32),
            pltpu.SemaphoreType.DMA,
            pltpu.SemaphoreType.DMA,
        ],
        compiler_params=pltpu.CompilerParams(needs_layout_passes=False),
    )(_sc_body)
    return f(s_key, c.reshape(B * NP), rand_idx)


def _finish_body(v_ref, p0_ref, o_ref):
    p0 = p0_ref[...]
    v = v_ref[...]
    e0 = jnp.exp(p0 / T)
    ev = jnp.exp(v / T)
    total = jnp.sum(e0) + jnp.sum(ev)
    z = total / (B * (K + 1)) * N_DATA
    o_ref[:, 0:1] = e0 / z
    o_ref[:, 1:] = ev / z


def _finish_stage(vals, p0):
    return pl.pallas_call(
        _finish_body,
        out_shape=jax.ShapeDtypeStruct((B, K + 1), jnp.float32),
    )(vals, p0)


def kernel(anchor_feature, pair_feature, membank_idx, threshold, memory_bank):
    thr_f = jnp.asarray(threshold, jnp.float32).reshape(1, 1)
    s_key, c, p0 = _mm_stage(anchor_feature, pair_feature, memory_bank, thr_f)

    rand_idx = jax.random.randint(jax.random.key(1234), (B, K), 0, N_DATA,
                                  dtype=jnp.int32)
    vals = _sc_topk(s_key, c, rand_idx)

    out = _finish_stage(vals, p0)
    return out.reshape(B, K + 1, 1)


# speedup vs baseline: 35.5801x; 1.1431x over previous
import functools
import math

import numpy as np

import jax
import jax.numpy as jnp
from jax import lax
from jax.experimental import pallas as pl
from jax.experimental.pallas import tpu as pltpu
from jax.experimental.pallas import tpu_sc as plsc

FEAT = 128
N_DATA = 100000
K = 1024
T = 0.07
B = 1024
TN = 2048
NP = 100352
SENT = -4.0

_I32 = jnp.int32

_NEG2_BITS = int(np.float32(-2.0).view(np.int32))
NEG2_KEY = _NEG2_BITS ^ ((_NEG2_BITS >> 31) & 0x7FFFFFFF)


def _f32_sort_key(x):
    b = jax.lax.bitcast_convert_type(x, _I32)
    return b ^ ((b >> 31) & jnp.int32(0x7FFFFFFF))


def _mm_body(a_ref, r_ref, m_ref, pair_ref, thr_ref, s_ref, c_ref, p0_ref):
    j = pl.program_id(0)
    a = a_ref[...]
    r = r_ref[...]
    m = m_ref[...]
    thr = thr_ref[0, 0]

    s = jax.lax.dot_general(a, r, (((1,), (0,)), ((), ())),
                            preferred_element_type=jnp.float32)
    c = jax.lax.dot_general(a, m, (((1,), (1,)), ((), ())),
                            preferred_element_type=jnp.float32)

    col = j * TN + jax.lax.broadcasted_iota(_I32, (B, TN), 1)
    s = jnp.where(s >= thr, jnp.float32(-2.0), s)
    s = jnp.where(col < N_DATA, s, jnp.float32(SENT))
    s_ref[...] = _f32_sort_key(s)
    c_ref[...] = c.reshape(B // 8, 8, TN // 128, 128).transpose(0, 2, 1, 3)

    @pl.when(j == 0)
    def _():
        p = pair_ref[...]
        p0_ref[...] = jnp.sum(a * p, axis=1, keepdims=True)


def _mm_stage(anchor, pair, memory_bank, thr_f):
    r = memory_bank.reshape(FEAT, N_DATA)
    grid = (NP // TN,)
    out_shapes = (
        jax.ShapeDtypeStruct((B, NP), _I32),
        jax.ShapeDtypeStruct((B // 8, NP // 128, 8, 128), jnp.float32),
        jax.ShapeDtypeStruct((B, 1), jnp.float32),
    )
    return pl.pallas_call(
        _mm_body,
        grid=grid,
        in_specs=[
            pl.BlockSpec((B, FEAT), lambda j: (0, 0)),
            pl.BlockSpec((FEAT, TN), lambda j: (0, j)),
            pl.BlockSpec((TN, FEAT), lambda j: (j, 0)),
            pl.BlockSpec((B, FEAT), lambda j: (0, 0)),
            pl.BlockSpec(memory_space=pltpu.SMEM),
        ],
        out_specs=(
            pl.BlockSpec((B, TN), lambda j: (0, j)),
            pl.BlockSpec((B // 8, TN // 128, 8, 128), lambda j: (0, j, 0, 0)),
            pl.BlockSpec((B, 1), lambda j: (0, 0)),
        ),
        out_shape=out_shapes,
    )(anchor, r, memory_bank, pair, thr_f)


NW = 32
RPW = B // NW
NBINS = 8192
BIN_SHIFT = 19
CAP = 2048
KINV_NEG2 = NEG2_KEY ^ 0x7FFFFFFF
_ONES16 = lambda: jnp.ones((16,), _I32)


def _sc_body(skey_hbm, cmat_hbm, rand_hbm, out_hbm,
             rowb, hist, ckA, ciA, ckB, ciB, offs, randb, flatb, valsb,
             sem, sem2):
    cid = lax.axis_index("c")
    sid = lax.axis_index("s")
    wid = sid * 2 + cid
    iota = lax.iota(_I32, 16)

    pltpu.async_copy(skey_hbm.at[wid * RPW], rowb, sem2)

    def do_row(i, _):
        r = wid * RPW + i
        pltpu.make_async_copy(skey_hbm.at[r], rowb, sem2).wait()
        pltpu.sync_copy(rand_hbm.at[r], randb)

        with jax.named_scope("zero_hist"):
            @plsc.parallel_loop(0, NBINS // 16, 1, unroll=8)
            def _(j):
                hist[pl.ds(j * 16, 16)] = jnp.zeros((16,), _I32)

        with jax.named_scope("hist"):
            @plsc.parallel_loop(0, NP // 16, 1, unroll=8)
            def _(v):
                k = rowb[pl.ds(v * 16, 16)]
                b = (k >> BIN_SHIFT) + (NBINS // 2)
                plsc.addupdate_scatter(hist, [b], _ONES16())

        target = jnp.int32(NP - K)

        with jax.named_scope("thresh_scan"):
            def scan_step(bchunk, carry):
                run, t = carry
                c = hist[pl.ds(bchunk * 16, 16)]
                cs = plsc.cumsum(c)
                cexc = cs - c + run
                cond = cexc <= target
                s = jnp.sum(cond.astype(_I32))
                t = jnp.where(s > 0, bchunk * 16 + s - 1, t)
                run = run + jnp.max(cs)
                return run, t
            _, t = lax.fori_loop(0, NBINS // 16, scan_step,
                                 (jnp.int32(0), jnp.int32(0)))
            thr_key = (t - NBINS // 2) << BIN_SHIFT

        with jax.named_scope("compact"):
            lane15 = jnp.full((16,), 15, _I32)

            def comp_step(g, off):
                ks, csms, mis = [], [], []
                for j in range(8):
                    k = rowb[pl.ds((g * 8 + j) * 16, 16)]
                    m = k >= thr_key
                    mi = m.astype(_I32)
                    ks.append(k)
                    mis.append(mi)
                    csms.append(plsc.cumsum(mi))
                for j in range(8):
                    pos = off + csms[j] - mis[j]
                    m2 = (mis[j] > 0) & (pos < CAP)
                    plsc.store_scatter(ckA, [pos],
                                       ks[j] ^ jnp.int32(0x7FFFFFFF),
                                       mask=m2)
                    plsc.store_scatter(ciA, [pos],
                                       (g * 8 + j) * 16 + iota, mask=m2)
                    off = off + jnp.take(csms[j], lane15)
                return off
            offv = lax.fori_loop(0, NP // 128, comp_step,
                                 jnp.zeros((16,), _I32))
            nc = jnp.minimum(jnp.max(offv), CAP)
            nv = (nc + 15) // 16

        @pl.when(i + 1 < RPW)
        def _():
            pltpu.async_copy(skey_hbm.at[r + 1], rowb, sem2)

        bufs = [(ckA, ciA, ckB, ciB), (ckB, ciB, ckA, ciA)]
        rscope = jax.named_scope("radix")
        rscope.__enter__()
        for p in range(5):
            ink, ini, outk, outi = bufs[p % 2]
            sh = 7 * p
            for bq in range(8):
                offs[pl.ds(bq * 16, 16)] = jnp.zeros((16,), _I32)

            @plsc.parallel_loop(0, nv, 1, unroll=4)
            def _(v, ink=ink, sh=sh):
                kk = ink[pl.ds(v * 16, 16)]
                gm = (v * 16 + iota) < nc
                d = lax.shift_right_logical(kk, sh) & 127
                plsc.addupdate_scatter(offs, [d], _ONES16(), mask=gm)

            run = jnp.int32(0)
            for bq in range(8):
                c = offs[pl.ds(bq * 16, 16)]
                cs = plsc.cumsum(c)
                offs[pl.ds(bq * 16, 16)] = cs - c + run
                run = run + jnp.max(cs)

            def perm_step(v, _, ink=ink, ini=ini, outk=outk, outi=outi, sh=sh):
                kk = ink[pl.ds(v * 16, 16)]
                vi = ini[pl.ds(v * 16, 16)]
                gm = (v * 16 + iota) < nc
                d = lax.shift_right_logical(kk, sh) & 127
                sk, sv, sm = plsc.sort_key_val(d, iota, mask=gm)
                prev = jnp.take(sk, jnp.maximum(iota - 1, 0))
                is_start = (iota == 0) | (sk != prev)
                base = plsc.cummax(jnp.where(is_start, iota, 0))
                rank = iota - base
                og = plsc.load_gather(offs, [sk], mask=sm)
                pos = og + rank
                kks = jnp.take(kk, sv)
                vis = jnp.take(vi, sv)
                plsc.store_scatter(outk, [pos], kks, mask=sm)
                plsc.store_scatter(outi, [pos], vis, mask=sm)
                plsc.addupdate_scatter(offs, [sk], _ONES16(), mask=sm)
                return 0
            lax.fori_loop(0, nv, perm_step, 0)
        rscope.__exit__(None, None, None)

        rbase = (r >> 3) * (NP // 128) * 1024 + (r & 7) * 128

        def sel_step(v, _):
            kv = ckB[pl.ds(v * 16, 16)]
            ivv = ciB[pl.ds(v * 16, 16)]
            rv = randb[pl.ds(v * 16, 16)]
            sel = kv != jnp.int32(KINV_NEG2)
            col = jnp.where(sel, ivv, rv)
            flatb[pl.ds(v * 16, 16)] = (rbase + ((col >> 7) << 10)
                                        + (col & 127))
            return 0
        lax.fori_loop(0, K // 16, sel_step, 0)

        pltpu.async_copy(cmat_hbm.at[flatb], valsb, sem).wait()
        pltpu.sync_copy(valsb, out_hbm.at[r])
        return 0

    lax.fori_loop(0, RPW, do_row, 0)


def _sc_topk(s_key, c4, rand_idx):
    mesh = plsc.VectorSubcoreMesh(core_axis_name="c", subcore_axis_name="s")
    f = functools.partial(
        pl.kernel,
        out_type=jax.ShapeDtypeStruct((B, K), jnp.float32),
        mesh=mesh,
        scratch_types=[
            pltpu.VMEM((NP,), _I32),
            pltpu.VMEM((NBINS,), _I32),
            pltpu.VMEM((CAP + 16,), _I32),
            pltpu.VMEM((CAP + 16,), _I32),
            pltpu.VMEM((CAP + 16,), _I32),
            pltpu.VMEM((CAP + 16,), _I32),
            pltpu.VMEM((128,), _I32),
            pltpu.VMEM((K,), _I32),
            pltpu.VMEM((K,), _I32),
            pltpu.VMEM((K,), jnp.float32),
            pltpu.SemaphoreType.DMA,
            pltpu.SemaphoreType.DMA,
        ],
        compiler_params=pltpu.CompilerParams(needs_layout_passes=False),
    )(_sc_body)
    return f(s_key, c4.reshape(B * NP), rand_idx)


def _finish_body(v_ref, p0_ref, o_ref):
    p0 = p0_ref[...]
    v = v_ref[...]
    e0 = jnp.exp(p0 / T)
    ev = jnp.exp(v / T)
    total = jnp.sum(e0) + jnp.sum(ev)
    z = total / (B * (K + 1)) * N_DATA
    o_ref[:, 0:1] = e0 / z
    o_ref[:, 1:] = ev / z


def _finish_stage(vals, p0):
    return pl.pallas_call(
        _finish_body,
        out_shape=jax.ShapeDtypeStruct((B, K + 1), jnp.float32),
    )(vals, p0)


def kernel(anchor_feature, pair_feature, membank_idx, threshold, memory_bank):
    thr_f = jnp.asarray(threshold, jnp.float32).reshape(1, 1)
    s_key, c, p0 = _mm_stage(anchor_feature, pair_feature, memory_bank, thr_f)

    rand_idx = jax.random.randint(jax.random.key(1234), (B, K), 0, N_DATA,
                                  dtype=jnp.int32)
    vals = _sc_topk(s_key, c, rand_idx)

    out = _finish_stage(vals, p0)
    return out.reshape(B, K + 1, 1)


# speedup vs baseline: 36.9971x; 1.0398x over previous
import functools
import math

import numpy as np

import jax
import jax.numpy as jnp
from jax import lax
from jax.experimental import pallas as pl
from jax.experimental.pallas import tpu as pltpu
from jax.experimental.pallas import tpu_sc as plsc

FEAT = 128
N_DATA = 100000
K = 1024
T = 0.07
B = 1024
TN = 2048
NP = 100352
SENT = -4.0

_I32 = jnp.int32

_NEG2_BITS = int(np.float32(-2.0).view(np.int32))
NEG2_KEY = _NEG2_BITS ^ ((_NEG2_BITS >> 31) & 0x7FFFFFFF)


def _f32_sort_key(x):
    b = jax.lax.bitcast_convert_type(x, _I32)
    return b ^ ((b >> 31) & jnp.int32(0x7FFFFFFF))


def _mm_body(a_ref, r_ref, m_ref, pair_ref, thr_ref, s_ref, c_ref, p0_ref):
    j = pl.program_id(0)
    a = a_ref[...]
    r = r_ref[...]
    m = m_ref[...]
    thr = thr_ref[0, 0]

    s = jax.lax.dot_general(a, r, (((1,), (0,)), ((), ())),
                            preferred_element_type=jnp.float32)
    c = jax.lax.dot_general(a, m, (((1,), (1,)), ((), ())),
                            preferred_element_type=jnp.float32)

    col = j * TN + jax.lax.broadcasted_iota(_I32, (B, TN), 1)
    s = jnp.where(s >= thr, jnp.float32(-2.0), s)
    s = jnp.where(col < N_DATA, s, jnp.float32(SENT))
    s_ref[...] = _f32_sort_key(s)
    c_ref[...] = c.reshape(B // 8, 8, TN // 128, 128).transpose(0, 2, 1, 3)

    @pl.when(j == 0)
    def _():
        p = pair_ref[...]
        p0_ref[...] = jnp.sum(a * p, axis=1, keepdims=True)


def _mm_stage(anchor, pair, memory_bank, thr_f):
    r = memory_bank.reshape(FEAT, N_DATA)
    grid = (NP // TN,)
    out_shapes = (
        jax.ShapeDtypeStruct((B, NP), _I32),
        jax.ShapeDtypeStruct((B // 8, NP // 128, 8, 128), jnp.float32),
        jax.ShapeDtypeStruct((B, 1), jnp.float32),
    )
    return pl.pallas_call(
        _mm_body,
        grid=grid,
        in_specs=[
            pl.BlockSpec((B, FEAT), lambda j: (0, 0)),
            pl.BlockSpec((FEAT, TN), lambda j: (0, j)),
            pl.BlockSpec((TN, FEAT), lambda j: (j, 0)),
            pl.BlockSpec((B, FEAT), lambda j: (0, 0)),
            pl.BlockSpec(memory_space=pltpu.SMEM),
        ],
        out_specs=(
            pl.BlockSpec((B, TN), lambda j: (0, j)),
            pl.BlockSpec((B // 8, TN // 128, 8, 128), lambda j: (0, j, 0, 0)),
            pl.BlockSpec((B, 1), lambda j: (0, 0)),
        ),
        out_shape=out_shapes,
    )(anchor, r, memory_bank, pair, thr_f)


NW = 32
RPW = B // NW
NBINS = 8192
BIN_SHIFT = 19
CAP = 2048
KINV_NEG2 = NEG2_KEY ^ 0x7FFFFFFF
_ONES16 = lambda: jnp.ones((16,), _I32)


def _sc_body(skey_hbm, cmat_hbm, rand_hbm, out_hbm,
             rowb, hist, ckA, ciA, ckB, ciB, offs, randb, flatb, valsb,
             sem, sem2):
    cid = lax.axis_index("c")
    sid = lax.axis_index("s")
    wid = sid * 2 + cid
    iota = lax.iota(_I32, 16)

    pltpu.async_copy(skey_hbm.at[wid * RPW], rowb, sem2)

    def do_row(i, _):
        r = wid * RPW + i
        pltpu.make_async_copy(skey_hbm.at[r], rowb, sem2).wait()
        pltpu.sync_copy(rand_hbm.at[r], randb)

        with jax.named_scope("zero_hist"):
            @plsc.parallel_loop(0, NBINS // 16, 1, unroll=8)
            def _(j):
                hist[pl.ds(j * 16, 16)] = jnp.zeros((16,), _I32)

        with jax.named_scope("hist"):
            @plsc.parallel_loop(0, NP // 16, 1, unroll=8)
            def _(v):
                k = rowb[pl.ds(v * 16, 16)]
                b = (k >> BIN_SHIFT) + (NBINS // 2)
                plsc.addupdate_scatter(hist, [b], _ONES16())

        target = jnp.int32(NP - K)

        with jax.named_scope("thresh_scan"):
            def scan_step(bchunk, carry):
                run, t = carry
                c = hist[pl.ds(bchunk * 16, 16)]
                cs = plsc.cumsum(c)
                cexc = cs - c + run
                cond = cexc <= target
                s = jnp.sum(cond.astype(_I32))
                t = jnp.where(s > 0, bchunk * 16 + s - 1, t)
                run = run + jnp.max(cs)
                return run, t
            _, t = lax.fori_loop(0, NBINS // 16, scan_step,
                                 (jnp.int32(0), jnp.int32(0)))
            thr_key = (t - NBINS // 2) << BIN_SHIFT

        with jax.named_scope("compact"):
            lane15 = jnp.full((16,), 15, _I32)

            def comp_step(g, off):
                ks, csms, mis = [], [], []
                for j in range(8):
                    k = rowb[pl.ds((g * 8 + j) * 16, 16)]
                    m = k >= thr_key
                    mi = m.astype(_I32)
                    ks.append(k)
                    mis.append(mi)
                    csms.append(plsc.cumsum(mi))
                for j in range(8):
                    pos = off + csms[j] - mis[j]
                    m2 = (mis[j] > 0) & (pos < CAP)
                    plsc.store_scatter(ckA, [pos],
                                       ks[j] ^ jnp.int32(0x7FFFFFFF),
                                       mask=m2)
                    plsc.store_scatter(ciA, [pos],
                                       (g * 8 + j) * 16 + iota, mask=m2)
                    off = off + jnp.take(csms[j], lane15)
                return off
            offv = lax.fori_loop(0, NP // 128, comp_step,
                                 jnp.zeros((16,), _I32))
            nc = jnp.minimum(jnp.max(offv), CAP)
            nv = (nc + 15) // 16

        @pl.when(i + 1 < RPW)
        def _():
            pltpu.async_copy(skey_hbm.at[r + 1], rowb, sem2)

        rscope = jax.named_scope("radix")
        rscope.__enter__()

        def range_step(v, carry):
            amin, amax = carry
            kk = ckA[pl.ds(v * 16, 16)] ^ jnp.int32(-0x80000000)
            gm = (v * 16 + iota) < nc
            amin = jnp.minimum(amin, jnp.where(gm, kk, jnp.int32(2**31 - 1)))
            amax = jnp.maximum(amax, jnp.where(gm, kk, jnp.int32(-2**31)))
            return amin, amax
        aminv, amaxv = lax.fori_loop(
            0, nv, range_step,
            (jnp.full((16,), 2**31 - 1, _I32), jnp.full((16,), -2**31, _I32)))
        kmin_x = jnp.min(aminv)
        krange = jnp.max(amaxv) - kmin_x

        bufs = [(ckA, ciA, ckB, ciB), (ckB, ciB, ckA, ciA)]
        for p in range(5):
            ink, ini, outk, outi = bufs[p % 2]
            sh = 7 * p
            if p == 0:
                active = jnp.bool_(True)
            else:
                active = lax.shift_right_logical(krange, sh) != 0

            @pl.when(active)
            def _(ink=ink, ini=ini, outk=outk, outi=outi, sh=sh):
                for bq in range(8):
                    offs[pl.ds(bq * 16, 16)] = jnp.zeros((16,), _I32)

                @plsc.parallel_loop(0, nv, 1, unroll=4)
                def _(v, ink=ink, sh=sh):
                    kk = ink[pl.ds(v * 16, 16)] ^ jnp.int32(-0x80000000)
                    gm = (v * 16 + iota) < nc
                    d = lax.shift_right_logical(kk - kmin_x, sh) & 127
                    plsc.addupdate_scatter(offs, [d], _ONES16(), mask=gm)

                run = jnp.int32(0)
                for bq in range(8):
                    c = offs[pl.ds(bq * 16, 16)]
                    cs = plsc.cumsum(c)
                    offs[pl.ds(bq * 16, 16)] = cs - c + run
                    run = run + jnp.max(cs)

                def perm_step(v, _, ink=ink, ini=ini, outk=outk, outi=outi,
                              sh=sh):
                    kk = ink[pl.ds(v * 16, 16)]
                    vi = ini[pl.ds(v * 16, 16)]
                    gm = (v * 16 + iota) < nc
                    kd = (kk ^ jnp.int32(-0x80000000)) - kmin_x
                    d = lax.shift_right_logical(kd, sh) & 127
                    sk, sv, sm = plsc.sort_key_val(d, iota, mask=gm)
                    prev = jnp.take(sk, jnp.maximum(iota - 1, 0))
                    is_start = (iota == 0) | (sk != prev)
                    base = plsc.cummax(jnp.where(is_start, iota, 0))
                    rank = iota - base
                    og = plsc.load_gather(offs, [sk], mask=sm)
                    pos = og + rank
                    kks = jnp.take(kk, sv)
                    vis = jnp.take(vi, sv)
                    plsc.store_scatter(outk, [pos], kks, mask=sm)
                    plsc.store_scatter(outi, [pos], vis, mask=sm)
                    plsc.addupdate_scatter(offs, [sk], _ONES16(), mask=sm)
                    return 0
                lax.fori_loop(0, nv, perm_step, 0)

            @pl.when(jnp.logical_not(active))
            def _(ink=ink, ini=ini, outk=outk, outi=outi):
                def copy_step(v, _, ink=ink, ini=ini, outk=outk, outi=outi):
                    outk[pl.ds(v * 16, 16)] = ink[pl.ds(v * 16, 16)]
                    outi[pl.ds(v * 16, 16)] = ini[pl.ds(v * 16, 16)]
                    return 0
                lax.fori_loop(0, nv, copy_step, 0)
        rscope.__exit__(None, None, None)

        rbase = (r >> 3) * (NP // 128) * 1024 + (r & 7) * 128

        def sel_step(v, _):
            kv = ckB[pl.ds(v * 16, 16)]
            ivv = ciB[pl.ds(v * 16, 16)]
            rv = randb[pl.ds(v * 16, 16)]
            sel = kv != jnp.int32(KINV_NEG2)
            col = jnp.where(sel, ivv, rv)
            flatb[pl.ds(v * 16, 16)] = (rbase + ((col >> 7) << 10)
                                        + (col & 127))
            return 0
        lax.fori_loop(0, K // 16, sel_step, 0)

        pltpu.async_copy(cmat_hbm.at[flatb], valsb, sem).wait()
        pltpu.sync_copy(valsb, out_hbm.at[r])
        return 0

    lax.fori_loop(0, RPW, do_row, 0)


def _sc_topk(s_key, c4, rand_idx):
    mesh = plsc.VectorSubcoreMesh(core_axis_name="c", subcore_axis_name="s")
    f = functools.partial(
        pl.kernel,
        out_type=jax.ShapeDtypeStruct((B, K), jnp.float32),
        mesh=mesh,
        scratch_types=[
            pltpu.VMEM((NP,), _I32),
            pltpu.VMEM((NBINS,), _I32),
            pltpu.VMEM((CAP + 16,), _I32),
            pltpu.VMEM((CAP + 16,), _I32),
            pltpu.VMEM((CAP + 16,), _I32),
            pltpu.VMEM((CAP + 16,), _I32),
            pltpu.VMEM((128,), _I32),
            pltpu.VMEM((K,), _I32),
            pltpu.VMEM((K,), _I32),
            pltpu.VMEM((K,), jnp.float32),
            pltpu.SemaphoreType.DMA,
            pltpu.SemaphoreType.DMA,
        ],
        compiler_params=pltpu.CompilerParams(needs_layout_passes=False),
    )(_sc_body)
    return f(s_key, c4.reshape(B * NP), rand_idx)


def _finish_body(v_ref, p0_ref, o_ref):
    p0 = p0_ref[...]
    v = v_ref[...]
    e0 = jnp.exp(p0 / T)
    ev = jnp.exp(v / T)
    total = jnp.sum(e0) + jnp.sum(ev)
    z = total / (B * (K + 1)) * N_DATA
    o_ref[:, 0:1] = e0 / z
    o_ref[:, 1:] = ev / z


def _finish_stage(vals, p0):
    return pl.pallas_call(
        _finish_body,
        out_shape=jax.ShapeDtypeStruct((B, K + 1), jnp.float32),
    )(vals, p0)


def kernel(anchor_feature, pair_feature, membank_idx, threshold, memory_bank):
    thr_f = jnp.asarray(threshold, jnp.float32).reshape(1, 1)
    s_key, c, p0 = _mm_stage(anchor_feature, pair_feature, memory_bank, thr_f)

    rand_idx = jax.random.randint(jax.random.key(1234), (B, K), 0, N_DATA,
                                  dtype=jnp.int32)
    vals = _sc_topk(s_key, c, rand_idx)

    out = _finish_stage(vals, p0)
    return out.reshape(B, K + 1, 1)
